# baseline (speedup 1.0000x reference)
import functools

import jax
import jax.numpy as jnp
from jax.experimental import pallas as pl
from jax.experimental.pallas import tpu as pltpu

B, S, D, DFF, E, TOP_K = 64, 1, 768, 3072, 16, 2
N = B * S
DH = D // 2
FH = DFF // 2
NB = 6
NCHUNK = 2 * E


def _moe_kernel(x_ref, wg_ref, bg_ref, w1_hbm, b1_ref, w2_hbm, b2_ref,
                alpha_ref, out_ref, w1_buf, w2_buf, w1_sem, w2_sem):
    def w1_copy(k):
        e, c = divmod(k, 2)
        return pltpu.make_async_copy(
            w1_hbm.at[e, pl.ds(c * DH, DH), :], w1_buf.at[k % NB],
            w1_sem.at[k % NB])

    def w2_copy(k):
        e, c = divmod(k, 2)
        return pltpu.make_async_copy(
            w2_hbm.at[e, pl.ds(c * FH, FH), :], w2_buf.at[k % NB],
            w2_sem.at[k % NB])

    for k in range(NB):
        w1_copy(k).start()
        w2_copy(k).start()

    x = x_ref[...]
    logits = jnp.dot(x, wg_ref[...], preferred_element_type=jnp.float32)
    logits = logits + bg_ref[0, :][None, :]
    m = jnp.max(logits, axis=-1, keepdims=True)
    ex = jnp.exp(logits - m)
    probs = ex / jnp.sum(ex, axis=-1, keepdims=True)
    ids = jax.lax.broadcasted_iota(jnp.int32, (N, E), 1)
    s1 = jnp.max(probs, axis=-1, keepdims=True)
    i1 = jnp.min(jnp.where(probs == s1, ids, E), axis=-1, keepdims=True)
    probs2 = jnp.where(ids == i1, -jnp.inf, probs)
    s2 = jnp.max(probs2, axis=-1, keepdims=True)
    i2 = jnp.min(jnp.where(probs2 == s2, ids, E), axis=-1, keepdims=True)
    w = jnp.where(ids == i1, s1, 0.0) + jnp.where(ids == i2, s2, 0.0)
    w = w * alpha_ref[0, :][None, :]

    def gelu(v):
        return 0.5 * v * (1.0 + jax.lax.erf(v * 0.7071067811865476))

    xl = x[:, :DH].astype(jnp.bfloat16)
    xh = x[:, DH:].astype(jnp.bfloat16)
    for e in range(E):
        k0, k1 = 2 * e, 2 * e + 1
        w1_copy(k0).wait()
        w1_copy(k1).wait()
        h = jnp.dot(xl, w1_buf[k0 % NB].astype(jnp.bfloat16),
                    preferred_element_type=jnp.float32)
        h = h + jnp.dot(xh, w1_buf[k1 % NB].astype(jnp.bfloat16),
                        preferred_element_type=jnp.float32)
        h = h + b1_ref[e][None, :]
        if k0 + NB < NCHUNK:
            w1_copy(k0 + NB).start()
        if k1 + NB < NCHUNK:
            w1_copy(k1 + NB).start()
        g = gelu(h)
        w2_copy(k0).wait()
        w2_copy(k1).wait()
        o = jnp.dot(g[:, :FH].astype(jnp.bfloat16),
                    w2_buf[k0 % NB].astype(jnp.bfloat16),
                    preferred_element_type=jnp.float32)
        o = o + jnp.dot(g[:, FH:].astype(jnp.bfloat16),
                        w2_buf[k1 % NB].astype(jnp.bfloat16),
                        preferred_element_type=jnp.float32)
        o = (o + b2_ref[e][None, :]) * w[:, e:e + 1]
        if k0 + NB < NCHUNK:
            w2_copy(k0 + NB).start()
        if k1 + NB < NCHUNK:
            w2_copy(k1 + NB).start()
        if e == 0:
            out_ref[...] = o
        else:
            out_ref[...] += o


@functools.partial(jax.jit, static_argnames=("interpret",))
def _moe(x, Wg, bg2, W1, b1, W2, b2, alpha2, interpret=False):
    return pl.pallas_call(
        _moe_kernel,
        in_specs=[
            pl.BlockSpec(memory_space=pltpu.VMEM),
            pl.BlockSpec(memory_space=pltpu.VMEM),
            pl.BlockSpec(memory_space=pltpu.VMEM),
            pl.BlockSpec(memory_space=pltpu.HBM),
            pl.BlockSpec(memory_space=pltpu.VMEM),
            pl.BlockSpec(memory_space=pltpu.HBM),
            pl.BlockSpec(memory_space=pltpu.VMEM),
            pl.BlockSpec(memory_space=pltpu.VMEM),
        ],
        out_specs=pl.BlockSpec(memory_space=pltpu.VMEM),
        out_shape=jax.ShapeDtypeStruct((N, D), jnp.float32),
        scratch_shapes=[
            pltpu.VMEM((NB, DH, DFF), jnp.float32),
            pltpu.VMEM((NB, FH, D), jnp.float32),
            pltpu.SemaphoreType.DMA((NB,)),
            pltpu.SemaphoreType.DMA((NB,)),
        ],
        interpret=interpret,
    )(x, Wg, bg2, W1, b1, W2, b2, alpha2)


def kernel(hidden_states, Wg, bg, W1, b1, W2, b2, alpha):
    b, s, d = hidden_states.shape
    x = hidden_states.reshape(-1, d)
    out = _moe(x, Wg, bg.reshape(1, E), W1, b1, W2, b2,
               alpha.reshape(1, E))
    return out.reshape(b, s, d)


# speedup vs baseline: 1.0437x; 1.0437x over previous
import functools

import jax
import jax.numpy as jnp
from jax.experimental import pallas as pl
from jax.experimental.pallas import tpu as pltpu

B, S, D, DFF, E, TOP_K = 64, 1, 768, 3072, 16, 2
N = B * S
DH = D // 2
FH = DFF // 2
NB = 4
NCHUNK = 2 * E


def _moe_kernel(x_ref, wg_ref, bg_ref, w1_hbm, b1_ref, w2_hbm, b2_ref,
                alpha_ref, out_ref, w1_buf, w2_buf, w1_sem, w2_sem):
    def w1_copy(k):
        e, c = divmod(k, 2)
        return pltpu.make_async_copy(
            w1_hbm.at[e, pl.ds(c * DH, DH), :], w1_buf.at[k % NB],
            w1_sem.at[k % NB])

    def w2_copy(k):
        e, c = divmod(k, 2)
        return pltpu.make_async_copy(
            w2_hbm.at[e, pl.ds(c * FH, FH), :], w2_buf.at[k % NB],
            w2_sem.at[k % NB])

    w1_copy(0).start()
    w1_copy(1).start()
    w2_copy(0).start()
    w2_copy(1).start()
    for k in range(2, NB):
        w1_copy(k).start()
        w2_copy(k).start()

    x = x_ref[...]
    logits = jnp.dot(x, wg_ref[...], preferred_element_type=jnp.float32)
    logits = logits + bg_ref[0, :][None, :]
    m = jnp.max(logits, axis=-1, keepdims=True)
    ex = jnp.exp(logits - m)
    probs = ex / jnp.sum(ex, axis=-1, keepdims=True)
    ids = jax.lax.broadcasted_iota(jnp.int32, (N, E), 1)
    s1 = jnp.max(probs, axis=-1, keepdims=True)
    i1 = jnp.min(jnp.where(probs == s1, ids, E), axis=-1, keepdims=True)
    probs2 = jnp.where(ids == i1, -jnp.inf, probs)
    s2 = jnp.max(probs2, axis=-1, keepdims=True)
    i2 = jnp.min(jnp.where(probs2 == s2, ids, E), axis=-1, keepdims=True)
    w = jnp.where(ids == i1, s1, 0.0) + jnp.where(ids == i2, s2, 0.0)
    w = w * alpha_ref[0, :][None, :]

    def gelu(v):
        return 0.5 * v * (1.0 + jax.lax.erf(v * 0.7071067811865476))

    xl = x[:, :DH].astype(jnp.bfloat16)
    xh = x[:, DH:].astype(jnp.bfloat16)
    for e in range(E):
        k0, k1 = 2 * e, 2 * e + 1
        w1_copy(k0).wait()
        h = jnp.dot(xl, w1_buf[k0 % NB].astype(jnp.bfloat16),
                    preferred_element_type=jnp.float32)
        if k0 + NB < NCHUNK:
            w1_copy(k0 + NB).start()
        w1_copy(k1).wait()
        h = h + jnp.dot(xh, w1_buf[k1 % NB].astype(jnp.bfloat16),
                        preferred_element_type=jnp.float32)
        if k1 + NB < NCHUNK:
            w1_copy(k1 + NB).start()
        h = h + b1_ref[e][None, :]
        gl = gelu(h[:, :FH]).astype(jnp.bfloat16)
        w2_copy(k0).wait()
        o = jnp.dot(gl, w2_buf[k0 % NB].astype(jnp.bfloat16),
                    preferred_element_type=jnp.float32)
        if k0 + NB < NCHUNK:
            w2_copy(k0 + NB).start()
        gh = gelu(h[:, FH:]).astype(jnp.bfloat16)
        w2_copy(k1).wait()
        o = o + jnp.dot(gh, w2_buf[k1 % NB].astype(jnp.bfloat16),
                        preferred_element_type=jnp.float32)
        if k1 + NB < NCHUNK:
            w2_copy(k1 + NB).start()
        o = (o + b2_ref[e][None, :]) * w[:, e:e + 1]
        if e == 0:
            out_ref[...] = o
        else:
            out_ref[...] += o


@functools.partial(jax.jit, static_argnames=("interpret",))
def _moe(x, Wg, bg2, W1, b1, W2, b2, alpha2, interpret=False):
    return pl.pallas_call(
        _moe_kernel,
        in_specs=[
            pl.BlockSpec(memory_space=pltpu.VMEM),
            pl.BlockSpec(memory_space=pltpu.VMEM),
            pl.BlockSpec(memory_space=pltpu.VMEM),
            pl.BlockSpec(memory_space=pltpu.HBM),
            pl.BlockSpec(memory_space=pltpu.VMEM),
            pl.BlockSpec(memory_space=pltpu.HBM),
            pl.BlockSpec(memory_space=pltpu.VMEM),
            pl.BlockSpec(memory_space=pltpu.VMEM),
        ],
        out_specs=pl.BlockSpec(memory_space=pltpu.VMEM),
        out_shape=jax.ShapeDtypeStruct((N, D), jnp.float32),
        scratch_shapes=[
            pltpu.VMEM((NB, DH, DFF), jnp.float32),
            pltpu.VMEM((NB, FH, D), jnp.float32),
            pltpu.SemaphoreType.DMA((NB,)),
            pltpu.SemaphoreType.DMA((NB,)),
        ],
        interpret=interpret,
    )(x, Wg, bg2, W1, b1, W2, b2, alpha2)


def kernel(hidden_states, Wg, bg, W1, b1, W2, b2, alpha):
    b, s, d = hidden_states.shape
    x = hidden_states.reshape(-1, d)
    out = _moe(x, Wg, bg.reshape(1, E), W1, b1, W2, b2,
               alpha.reshape(1, E))
    return out.reshape(b, s, d)


# speedup vs baseline: 1.0564x; 1.0121x over previous
import functools

import jax
import jax.numpy as jnp
from jax.experimental import pallas as pl
from jax.experimental.pallas import tpu as pltpu

B, S, D, DFF, E, TOP_K = 64, 1, 768, 3072, 16, 2
N = B * S
DH = D // 2
FH = DFF // 2
FQ = DFF // 4
NB1 = 5
NB2 = 8
NC1 = 2 * E
NC2 = 4 * E


def _moe_kernel(x_ref, wg_ref, bg_ref, w1_hbm, b1_ref, w2_hbm, b2_ref,
                alpha_ref, out_ref, w1_buf, w2_buf, w1_sem, w2_sem):
    def w1_copy(k):
        e, c = divmod(k, 2)
        return pltpu.make_async_copy(
            w1_hbm.at[e, pl.ds(c * DH, DH), :], w1_buf.at[k % NB1],
            w1_sem.at[k % NB1])

    def w2_copy(k):
        e, c = divmod(k, 4)
        return pltpu.make_async_copy(
            w2_hbm.at[e, pl.ds(c * FQ, FQ), :], w2_buf.at[k % NB2],
            w2_sem.at[k % NB2])

    w1_copy(0).start()
    w1_copy(1).start()
    for k in range(4):
        w2_copy(k).start()
    for k in range(2, NB1):
        w1_copy(k).start()
    for k in range(4, NB2):
        w2_copy(k).start()

    x = x_ref[...]
    logits = jnp.dot(x, wg_ref[...], preferred_element_type=jnp.float32)
    logits = logits + bg_ref[0, :][None, :]
    m = jnp.max(logits, axis=-1, keepdims=True)
    ex = jnp.exp(logits - m)
    probs = ex / jnp.sum(ex, axis=-1, keepdims=True)
    ids = jax.lax.broadcasted_iota(jnp.int32, (N, E), 1)
    s1 = jnp.max(probs, axis=-1, keepdims=True)
    i1 = jnp.min(jnp.where(probs == s1, ids, E), axis=-1, keepdims=True)
    probs2 = jnp.where(ids == i1, -jnp.inf, probs)
    s2 = jnp.max(probs2, axis=-1, keepdims=True)
    i2 = jnp.min(jnp.where(probs2 == s2, ids, E), axis=-1, keepdims=True)
    w = jnp.where(ids == i1, s1, 0.0) + jnp.where(ids == i2, s2, 0.0)
    w = w * alpha_ref[0, :][None, :]

    def gelu(v):
        return 0.5 * v * (1.0 + jax.lax.erf(v * 0.7071067811865476))

    xl = x[:, :DH].astype(jnp.bfloat16)
    xh = x[:, DH:].astype(jnp.bfloat16)
    for e in range(E):
        k0, k1 = 2 * e, 2 * e + 1
        w1_copy(k0).wait()
        h = jnp.dot(xl, w1_buf[k0 % NB1].astype(jnp.bfloat16),
                    preferred_element_type=jnp.float32)
        if k0 + NB1 < NC1:
            w1_copy(k0 + NB1).start()
        w1_copy(k1).wait()
        h = h + jnp.dot(xh, w1_buf[k1 % NB1].astype(jnp.bfloat16),
                        preferred_element_type=jnp.float32)
        if k1 + NB1 < NC1:
            w1_copy(k1 + NB1).start()
        h = h + b1_ref[e][None, :]
        o = None
        for q in range(4):
            kq = 4 * e + q
            gq = gelu(h[:, q * FQ:(q + 1) * FQ]).astype(jnp.bfloat16)
            w2_copy(kq).wait()
            oq = jnp.dot(gq, w2_buf[kq % NB2].astype(jnp.bfloat16),
                         preferred_element_type=jnp.float32)
            o = oq if o is None else o + oq
            if kq + NB2 < NC2:
                w2_copy(kq + NB2).start()
        o = (o + b2_ref[e][None, :]) * w[:, e:e + 1]
        if e == 0:
            out_ref[...] = o
        else:
            out_ref[...] += o


@functools.partial(jax.jit, static_argnames=("interpret",))
def _moe(x, Wg, bg2, W1, b1, W2, b2, alpha2, interpret=False):
    return pl.pallas_call(
        _moe_kernel,
        in_specs=[
            pl.BlockSpec(memory_space=pltpu.VMEM),
            pl.BlockSpec(memory_space=pltpu.VMEM),
            pl.BlockSpec(memory_space=pltpu.VMEM),
            pl.BlockSpec(memory_space=pltpu.HBM),
            pl.BlockSpec(memory_space=pltpu.VMEM),
            pl.BlockSpec(memory_space=pltpu.HBM),
            pl.BlockSpec(memory_space=pltpu.VMEM),
            pl.BlockSpec(memory_space=pltpu.VMEM),
        ],
        out_specs=pl.BlockSpec(memory_space=pltpu.VMEM),
        out_shape=jax.ShapeDtypeStruct((N, D), jnp.float32),
        scratch_shapes=[
            pltpu.VMEM((NB1, DH, DFF), jnp.float32),
            pltpu.VMEM((NB2, FQ, D), jnp.float32),
            pltpu.SemaphoreType.DMA((NB1,)),
            pltpu.SemaphoreType.DMA((NB2,)),
        ],
        interpret=interpret,
    )(x, Wg, bg2, W1, b1, W2, b2, alpha2)


def kernel(hidden_states, Wg, bg, W1, b1, W2, b2, alpha):
    b, s, d = hidden_states.shape
    x = hidden_states.reshape(-1, d)
    out = _moe(x, Wg, bg.reshape(1, E), W1, b1, W2, b2,
               alpha.reshape(1, E))
    return out.reshape(b, s, d)
